# baseline (speedup 1.0000x reference)
import jax
import jax.numpy as jnp
from jax.experimental import pallas as pl
from jax.experimental.pallas import tpu as pltpu

N_ELEMS = 16384 * 12
VROWS = 1536
GRID = 8
RB = VROWS // GRID


def _block_body(lg_ref, lb_ref, og_ref, out_ref, acc_ref):
    i = pl.program_id(0)

    @pl.when(i == 0)
    def _init():
        acc_ref[...] = jnp.zeros_like(acc_ref)

    lg = lg_ref[...]
    lb = lb_ref[...]
    og = og_ref[...]
    l = jnp.where(lg > 0.0, og, 0.0)
    tm = lb > 0.001
    diff = l - lb
    safe = jnp.where(tm, lb, 1.0)
    pe = jnp.where(tm, diff * diff / safe, l * l)

    part = jnp.zeros((8, 128), jnp.float32)
    for r in range(RB // 8):
        part = part + pe[8 * r:8 * r + 8, :]
    acc_ref[...] += part

    @pl.when(i == GRID - 1)
    def _finish():
        total = jnp.sum(acc_ref[...]) * (1.0 / N_ELEMS)
        out_ref[...] = total[None, None]


_loss_call = pl.pallas_call(
    _block_body,
    grid=(GRID,),
    in_specs=[
        pl.BlockSpec((RB, 128), lambda i: (i, 0)),
        pl.BlockSpec((RB, 128), lambda i: (i, 0)),
        pl.BlockSpec((RB, 128), lambda i: (i, 0)),
    ],
    out_specs=pl.BlockSpec((1, 1), lambda i: (0, 0)),
    out_shape=jax.ShapeDtypeStruct((1, 1), jnp.float32),
    scratch_shapes=[pltpu.VMEM((8, 128), jnp.float32)],
)


def kernel(logits, label, og_x):
    og_c = og_x[:, :12, :]
    lg = logits.reshape(VROWS, 128)
    lb = label.reshape(VROWS, 128)
    og = og_c.reshape(VROWS, 128)
    return _loss_call(lg, lb, og).reshape(())


# speedup vs baseline: 7.6016x; 7.6016x over previous
import jax
import jax.numpy as jnp
from jax.experimental import pallas as pl
from jax.experimental.pallas import tpu as pltpu

N_ELEMS = 16384 * 12
GRID = 8
BS = 128 // GRID


def _block_body(lg_ref, lb_ref, og_ref, out_ref, acc_ref):
    i = pl.program_id(0)

    @pl.when(i == 0)
    def _init():
        acc_ref[...] = jnp.zeros_like(acc_ref)

    lg = lg_ref[...]
    lb = lb_ref[...]
    og = og_ref[...]
    l = jnp.where(lg > 0.0, og, 0.0)
    tm = lb > 0.001
    diff = l - lb
    safe = jnp.where(tm, lb, 1.0)
    pe = jnp.where(tm, diff * diff / safe, l * l)

    part = jnp.zeros((8, 128), jnp.float32)
    for j in range(12):
        for b in range(BS // 8):
            part = part + pe[j, 8 * b:8 * b + 8, :]
    acc_ref[...] += part

    @pl.when(i == GRID - 1)
    def _finish():
        total = jnp.sum(acc_ref[...]) * (1.0 / N_ELEMS)
        out_ref[...] = total[None, None]


_loss_call = pl.pallas_call(
    _block_body,
    grid=(GRID,),
    in_specs=[
        pl.BlockSpec((12, BS, 128), lambda i: (0, i, 0)),
        pl.BlockSpec((12, BS, 128), lambda i: (0, i, 0)),
        pl.BlockSpec((12, BS, 128), lambda i: (0, i, 0)),
    ],
    out_specs=pl.BlockSpec((1, 1), lambda i: (0, 0)),
    out_shape=jax.ShapeDtypeStruct((1, 1), jnp.float32),
    scratch_shapes=[pltpu.VMEM((8, 128), jnp.float32)],
    compiler_params=pltpu.CompilerParams(
        allow_input_fusion=[True, True, True]),
)


def kernel(logits, label, og_x):
    lg = logits.reshape(16384, 12).T.reshape(12, 128, 128)
    lb = label.reshape(16384, 12).T.reshape(12, 128, 128)
    og = og_x.reshape(16384, 24).T.reshape(24, 128, 128)
    return _loss_call(lg, lb, og).reshape(())


# speedup vs baseline: 7.7163x; 1.0151x over previous
import jax
import jax.numpy as jnp
from jax.experimental import pallas as pl
from jax.experimental.pallas import tpu as pltpu

N_ELEMS = 16384 * 12
GRID = 4
BS = 128 // GRID


def _block_body(lg_ref, lb_ref, og_ref, out_ref, acc_ref):
    i = pl.program_id(0)

    @pl.when(i == 0)
    def _init():
        acc_ref[...] = jnp.zeros_like(acc_ref)

    lg = lg_ref[...]
    lb = lb_ref[...]
    og = og_ref[...]
    l = jnp.where(lg > 0.0, og, 0.0)
    tm = lb > 0.001
    diff = l - lb
    safe = jnp.where(tm, lb, 1.0)
    pe = jnp.where(tm, diff * diff / safe, l * l)

    part = jnp.zeros((8, 128), jnp.float32)
    for j in range(12):
        for b in range(BS // 8):
            part = part + pe[j, 8 * b:8 * b + 8, :]
    acc_ref[...] += part

    @pl.when(i == GRID - 1)
    def _finish():
        total = jnp.sum(acc_ref[...]) * (1.0 / N_ELEMS)
        out_ref[...] = total[None, None]


_loss_call = pl.pallas_call(
    _block_body,
    grid=(GRID,),
    in_specs=[
        pl.BlockSpec((12, BS, 128), lambda i: (0, i, 0)),
        pl.BlockSpec((12, BS, 128), lambda i: (0, i, 0)),
        pl.BlockSpec((12, BS, 128), lambda i: (0, i, 0)),
    ],
    out_specs=pl.BlockSpec((1, 1), lambda i: (0, 0)),
    out_shape=jax.ShapeDtypeStruct((1, 1), jnp.float32),
    scratch_shapes=[pltpu.VMEM((8, 128), jnp.float32)],
    compiler_params=pltpu.CompilerParams(
        allow_input_fusion=[True, True, True]),
)


def kernel(logits, label, og_x):
    lg = logits.reshape(16384, 12).T.reshape(12, 128, 128)
    lb = label.reshape(16384, 12).T.reshape(12, 128, 128)
    og = og_x.reshape(16384, 24).T.reshape(24, 128, 128)
    return _loss_call(lg, lb, og).reshape(())


# speedup vs baseline: 7.7637x; 1.0061x over previous
import jax
import jax.numpy as jnp
from jax.experimental import pallas as pl
from jax.experimental.pallas import tpu as pltpu

N_ELEMS = 16384 * 12
GRID = 2
BS = 128 // GRID


def _block_body(lg_ref, lb_ref, og_ref, out_ref, acc_ref):
    i = pl.program_id(0)

    @pl.when(i == 0)
    def _init():
        acc_ref[...] = jnp.zeros_like(acc_ref)

    lg = lg_ref[...]
    lb = lb_ref[...]
    og = og_ref[...]
    l = jnp.where(lg > 0.0, og, 0.0)
    tm = lb > 0.001
    diff = l - lb
    safe = jnp.where(tm, lb, 1.0)
    pe = jnp.where(tm, diff * diff / safe, l * l)

    part = jnp.zeros((8, 128), jnp.float32)
    for j in range(12):
        for b in range(BS // 8):
            part = part + pe[j, 8 * b:8 * b + 8, :]
    acc_ref[...] += part

    @pl.when(i == GRID - 1)
    def _finish():
        total = jnp.sum(acc_ref[...]) * (1.0 / N_ELEMS)
        out_ref[...] = total[None, None]


_loss_call = pl.pallas_call(
    _block_body,
    grid=(GRID,),
    in_specs=[
        pl.BlockSpec((12, BS, 128), lambda i: (0, i, 0)),
        pl.BlockSpec((12, BS, 128), lambda i: (0, i, 0)),
        pl.BlockSpec((12, BS, 128), lambda i: (0, i, 0)),
    ],
    out_specs=pl.BlockSpec((1, 1), lambda i: (0, 0)),
    out_shape=jax.ShapeDtypeStruct((1, 1), jnp.float32),
    scratch_shapes=[pltpu.VMEM((8, 128), jnp.float32)],
    compiler_params=pltpu.CompilerParams(
        allow_input_fusion=[True, True, True]),
)


def kernel(logits, label, og_x):
    lg = logits.reshape(16384, 12).T.reshape(12, 128, 128)
    lb = label.reshape(16384, 12).T.reshape(12, 128, 128)
    og = og_x.reshape(16384, 24).T.reshape(24, 128, 128)
    return _loss_call(lg, lb, og).reshape(())
